# baseline (speedup 1.0000x reference)
import functools

import jax
import jax.numpy as jnp
from jax import lax
from jax.experimental import pallas as pl
from jax.experimental.pallas import tpu as pltpu
from jax.experimental.pallas import tpu_sc as plsc

_N = 10000
_D = 128
_G = 200
_NC = 2
_NS = 16
_NW = _NC * _NS
_K = 128
_NP = 10240
_RPS = _NP // _NS
_CPT = 80
_EPT = _CPT * _K
_EP = _NW * _EPT


def _deg_body(dst_hbm, out_hbm, didx, ones, zbuf, acc, sem, ssem):
    c = lax.axis_index("c")
    s = lax.axis_index("s")
    t = c * _NS + s

    def _fill(r, _):
        ones[r, :] = jnp.full((16,), 1.0, jnp.float32)
        zbuf[r, :] = jnp.zeros((16,), jnp.float32)
        return 0
    lax.fori_loop(0, _K, _fill, 0)

    pltpu.sync_copy(dst_hbm.at[pl.ds(t * _CPT, _CPT)], didx)

    base_r = s * _RPS
    for q in range(_RPS // _K):
        pltpu.sync_copy(zbuf, acc.at[pl.ds(base_r + q * _K, _K)])
    plsc.subcore_barrier()

    kblk = 20

    def _blk(kk, _):
        def _fire(i, _):
            pltpu.async_copy(ones, acc.at[didx.at[kk * kblk + i]], ssem,
                             add=True)
            return 0
        lax.fori_loop(0, kblk, _fire, 0)

        def _drain(i, _):
            pltpu.make_async_copy(ones, acc.at[didx.at[0]], ssem).wait()
            return 0
        lax.fori_loop(0, kblk, _drain, 0)
        return 0
    lax.fori_loop(0, _CPT // kblk, _blk, 0)
    plsc.subcore_barrier()

    pltpu.sync_copy(acc.at[pl.ds(base_r, _RPS)],
                    out_hbm.at[c, pl.ds(base_r, _RPS)])


def _make_deg_call():
    mesh = plsc.VectorSubcoreMesh(core_axis_name="c", subcore_axis_name="s")
    return pl.kernel(
        _deg_body,
        out_type=jax.ShapeDtypeStruct((_NC, _NP, 16), jnp.float32),
        mesh=mesh,
        scratch_types=[
            pltpu.VMEM((_CPT, _K), jnp.int32),
            pltpu.VMEM((_K, 16), jnp.float32),
            pltpu.VMEM((_K, 16), jnp.float32),
            pltpu.VMEM_SHARED((_NP, 16), jnp.float32),
            pltpu.SemaphoreType.DMA,
            pltpu.SemaphoreType.DMA,
        ],
        name="sc_degree_count",
    )


_DH = _D // 2


_GRP = 4
_NGRP = _CPT // _GRP


def _scatter_body(hsa_hbm, hsb_hbm, src_hbm, dst_hbm, out_hbm, sidx, didx,
                  r0, r1, r2, r3, r4, r5, r6, r7, acc, gsem, ssem):
    c = lax.axis_index("c")
    s = lax.axis_index("s")
    t = c * _NS + s
    rows = (r0, r1, r2, r3, r4, r5, r6, r7)
    rows0 = r0
    base_r = s * _RPS

    def _fill_zero():
        def _f(r, _):
            for q in range(_DH // 16):
                rows0[r, pl.ds(q * 16, 16)] = jnp.zeros((16,), jnp.float32)
            return 0
        lax.fori_loop(0, _K, _f, 0)

    def _zero_acc():
        for q in range(_RPS // _K):
            pltpu.sync_copy(rows0, acc.at[pl.ds(base_r + q * _K, _K)])

    _fill_zero()
    _zero_acc()
    pltpu.sync_copy(src_hbm.at[pl.ds(t * _CPT, _CPT)], sidx)
    pltpu.sync_copy(dst_hbm.at[pl.ds(t * _CPT, _CPT)], didx)
    plsc.subcore_barrier()

    def _phase(hs_hbm, ph):
        def _fire_g(group, s0):
            for i in range(_GRP):
                pltpu.async_copy(hs_hbm.at[sidx.at[group * _GRP + i]],
                                 rows[s0 * _GRP + i], gsem)

        def _drain_g():
            for _i in range(_GRP):
                pltpu.make_async_copy(hs_hbm.at[sidx.at[0]], rows0,
                                      gsem).wait()

        def _fire_s(group, s0):
            for i in range(_GRP):
                pltpu.async_copy(rows[s0 * _GRP + i],
                                 acc.at[didx.at[group * _GRP + i]], ssem,
                                 add=True)

        def _drain_s():
            for _i in range(_GRP):
                pltpu.make_async_copy(rows0, acc.at[didx.at[0]], ssem).wait()

        def _pair(j, last):
            a = 2 * j
            b = a + 1
            _drain_g()
            _fire_s(a, 0)
            _fire_g(b, 1)
            _drain_s()
            _drain_g()
            _fire_s(b, 1)
            if not last:
                _fire_g(a + 2, 0)
            _drain_s()

        _fire_g(0, 0)

        def _body(j, _):
            _pair(j, False)
            return 0
        lax.fori_loop(0, _NGRP // 2 - 1, _body, 0)
        _pair(_NGRP // 2 - 1, True)
        plsc.subcore_barrier()
        pltpu.sync_copy(acc.at[pl.ds(base_r, _RPS)],
                        out_hbm.at[c, ph, pl.ds(base_r, _RPS)])

    _phase(hsa_hbm, 0)
    _fill_zero()
    _zero_acc()
    plsc.subcore_barrier()
    _phase(hsb_hbm, 1)


def _make_scatter_call():
    mesh = plsc.VectorSubcoreMesh(core_axis_name="c", subcore_axis_name="s")
    return pl.kernel(
        _scatter_body,
        out_type=jax.ShapeDtypeStruct((_NC, 2, _NP, _DH), jnp.float32),
        mesh=mesh,
        scratch_types=[
            pltpu.VMEM((_CPT, _K), jnp.int32),
            pltpu.VMEM((_CPT, _K), jnp.int32),
            pltpu.VMEM((_K, _DH), jnp.float32),
            pltpu.VMEM((_K, _DH), jnp.float32),
            pltpu.VMEM((_K, _DH), jnp.float32),
            pltpu.VMEM((_K, _DH), jnp.float32),
            pltpu.VMEM((_K, _DH), jnp.float32),
            pltpu.VMEM((_K, _DH), jnp.float32),
            pltpu.VMEM((_K, _DH), jnp.float32),
            pltpu.VMEM((_K, _DH), jnp.float32),
            pltpu.VMEM_SHARED((_NP, _DH), jnp.float32),
            pltpu.SemaphoreType.DMA,
            pltpu.SemaphoreType.DMA,
        ],
        compiler_params=pltpu.CompilerParams(use_tc_tiling_on_sc=False),
        name="sc_edge_scatter_add",
    )


def _psum_hs(p_ref, hsa_ref, hsb_ref):
    pa = p_ref[0, 0, : _N, :] + p_ref[1, 0, : _N, :] + hsa_ref[...]
    pb = p_ref[0, 1, : _N, :] + p_ref[1, 1, : _N, :] + hsb_ref[...]
    return jnp.concatenate([pa, pb], axis=1)


def _pre_body(x_ref, wx_ref, bx_ref, wg_ref, hw_ref):
    h0 = jnp.dot(x_ref[...], wx_ref[...],
                 preferred_element_type=jnp.float32) + bx_ref[...]
    hw_ref[...] = jnp.dot(h0, wg_ref[...], preferred_element_type=jnp.float32)


_pre_call = pl.pallas_call(
    _pre_body,
    out_shape=jax.ShapeDtypeStruct((_N, _D), jnp.float32),
)


def _embed_body(hw_ref, degs_ref, hsa_ref, hsb_ref, dinv_ref):
    deg = degs_ref[0, : _N, 0:1] + degs_ref[1, : _N, 0:1] + 1.0
    dinv = lax.rsqrt(deg)
    hs = hw_ref[...] * dinv
    hsa_ref[...] = hs[:, : _DH]
    hsb_ref[...] = hs[:, _DH:]
    dinv_ref[...] = dinv


_embed_call = pl.pallas_call(
    _embed_body,
    out_shape=[
        jax.ShapeDtypeStruct((_N, _DH), jnp.float32),
        jax.ShapeDtypeStruct((_N, _DH), jnp.float32),
        jax.ShapeDtypeStruct((_N, 1), jnp.float32),
    ],
)


def _mid_body(p_ref, hsa_ref, hsb_ref, dinv_ref, bg_ref, g_ref, be_ref,
              wn_ref, outa_ref, outb_ref):
    dinv = dinv_ref[...]
    t = dinv * _psum_hs(p_ref, hsa_ref, hsb_ref) + bg_ref[...]
    m = jnp.mean(t, axis=0, keepdims=True)
    d = t - m
    v = jnp.mean(d * d, axis=0, keepdims=True)
    h = d * lax.rsqrt(v + 1e-5) * g_ref[...] + be_ref[...]
    h = jnp.maximum(h, 0.0)
    hs = jnp.dot(h, wn_ref[...], preferred_element_type=jnp.float32) * dinv
    outa_ref[...] = hs[:, : _DH]
    outb_ref[...] = hs[:, _DH:]


_mid_call = pl.pallas_call(
    _mid_body,
    out_shape=[
        jax.ShapeDtypeStruct((_N, _DH), jnp.float32),
        jax.ShapeDtypeStruct((_N, _DH), jnp.float32),
    ],
)


def _head_body(p_ref, hsa_ref, hsb_ref, dinv_ref, bg_ref, g_ref, be_ref,
               batch_ref, wf0_ref, bf0_ref, wf1_ref, bf1_ref, wf2_ref,
               bf2_ref, out_ref):
    t = dinv_ref[...] * _psum_hs(p_ref, hsa_ref, hsb_ref) + bg_ref[...]
    m = jnp.mean(t, axis=0, keepdims=True)
    d = t - m
    v = jnp.mean(d * d, axis=0, keepdims=True)
    h = d * lax.rsqrt(v + 1e-5) * g_ref[...] + be_ref[...]

    gids = lax.broadcasted_iota(jnp.int32, (_G, 1), 0)
    onehot = (batch_ref[...] == gids).astype(jnp.float32)
    sums = jnp.dot(onehot, h, preferred_element_type=jnp.float32)
    cnt = jnp.sum(onehot, axis=1, keepdims=True)
    o = sums / jnp.maximum(cnt, 1.0)

    o = jnp.maximum(jnp.dot(o, wf0_ref[...],
                            preferred_element_type=jnp.float32) + bf0_ref[...], 0.0)
    o = jnp.maximum(jnp.dot(o, wf1_ref[...],
                            preferred_element_type=jnp.float32) + bf1_ref[...], 0.0)
    out_ref[...] = jnp.dot(o, wf2_ref[...],
                           preferred_element_type=jnp.float32) + bf2_ref[...]


_head_call = pl.pallas_call(
    _head_body,
    out_shape=jax.ShapeDtypeStruct((_G, 1), jnp.float32),
)


def kernel(x, edge_index, edge_attr, batch, Wx, bx, Wg0, bg0, g0, be0,
           Wg1, bg1, g1, be1, Wg2, bg2, g2, be2, Wf0, bf0, Wf1, bf1, Wf2, bf2):
    e = edge_index.shape[1]
    src = edge_index[0].astype(jnp.int32)
    dst = edge_index[1].astype(jnp.int32)
    pad = _EP - e
    pidx = jnp.arange(pad, dtype=jnp.int32)
    src_p = jnp.concatenate([src, pidx % _N])
    dst_p = jnp.concatenate([dst, _N + pidx % (_NP - _N)])
    src_p = src_p.reshape(_NW * _CPT, _K)
    dst_p = dst_p.reshape(_NW * _CPT, _K)

    deg_call = _make_deg_call()
    scatter_call = _make_scatter_call()

    hw0 = _pre_call(x, Wx, bx, Wg0)
    degs = deg_call(dst_p)
    hs0a, hs0b, dinv = _embed_call(hw0, degs)
    p0 = scatter_call(hs0a, hs0b, src_p, dst_p)
    hs1a, hs1b = _mid_call(p0, hs0a, hs0b, dinv, bg0, g0, be0, Wg1)
    p1 = scatter_call(hs1a, hs1b, src_p, dst_p)
    hs2a, hs2b = _mid_call(p1, hs1a, hs1b, dinv, bg1, g1, be1, Wg2)
    p2 = scatter_call(hs2a, hs2b, src_p, dst_p)
    batch2d = batch.astype(jnp.int32).reshape(1, _N)
    return _head_call(p2, hs2a, hs2b, dinv, bg2, g2, be2, batch2d,
                      Wf0, bf0, Wf1, bf1, Wf2, bf2)


# speedup vs baseline: 1.0028x; 1.0028x over previous
import functools

import jax
import jax.numpy as jnp
from jax import lax
from jax.experimental import pallas as pl
from jax.experimental.pallas import tpu as pltpu
from jax.experimental.pallas import tpu_sc as plsc

_N = 10000
_D = 128
_G = 200
_NC = 2
_NS = 16
_NW = _NC * _NS
_K = 128
_NP = 10240
_RPS = _NP // _NS
_CPT = 80
_EPT = _CPT * _K
_EP = _NW * _EPT


def _deg_body(dst_hbm, out_hbm, didx, ones, zbuf, acc, sem, ssem):
    c = lax.axis_index("c")
    s = lax.axis_index("s")
    t = c * _NS + s

    def _fill(r, _):
        ones[r, :] = jnp.full((16,), 1.0, jnp.float32)
        zbuf[r, :] = jnp.zeros((16,), jnp.float32)
        return 0
    lax.fori_loop(0, _K, _fill, 0)

    pltpu.sync_copy(dst_hbm.at[pl.ds(t * _CPT, _CPT)], didx)

    base_r = s * _RPS
    for q in range(_RPS // _K):
        pltpu.sync_copy(zbuf, acc.at[pl.ds(base_r + q * _K, _K)])
    plsc.subcore_barrier()

    kblk = 20

    def _blk(kk, _):
        def _fire(i, _):
            pltpu.async_copy(ones, acc.at[didx.at[kk * kblk + i]], ssem,
                             add=True)
            return 0
        lax.fori_loop(0, kblk, _fire, 0)

        def _drain(i, _):
            pltpu.make_async_copy(ones, acc.at[didx.at[0]], ssem).wait()
            return 0
        lax.fori_loop(0, kblk, _drain, 0)
        return 0
    lax.fori_loop(0, _CPT // kblk, _blk, 0)
    plsc.subcore_barrier()

    pltpu.sync_copy(acc.at[pl.ds(base_r, _RPS)],
                    out_hbm.at[c, pl.ds(base_r, _RPS)])


def _make_deg_call():
    mesh = plsc.VectorSubcoreMesh(core_axis_name="c", subcore_axis_name="s")
    return pl.kernel(
        _deg_body,
        out_type=jax.ShapeDtypeStruct((_NC, _NP, 16), jnp.float32),
        mesh=mesh,
        scratch_types=[
            pltpu.VMEM((_CPT, _K), jnp.int32),
            pltpu.VMEM((_K, 16), jnp.float32),
            pltpu.VMEM((_K, 16), jnp.float32),
            pltpu.VMEM_SHARED((_NP, 16), jnp.float32),
            pltpu.SemaphoreType.DMA,
            pltpu.SemaphoreType.DMA,
        ],
        name="sc_degree_count",
    )


_DH = _D // 2


_GRP = 4
_NGRP = _CPT // _GRP


def _scatter_body(hsa_hbm, hsb_hbm, src_hbm, dst_hbm, out_hbm, sidx, didx,
                  r0, r1, r2, r3, r4, r5, r6, r7, acc, gsem, ssem):
    c = lax.axis_index("c")
    s = lax.axis_index("s")
    t = c * _NS + s
    rows = (r0, r1, r2, r3, r4, r5, r6, r7)
    rows0 = r0
    base_r = s * _RPS

    def _fill_zero():
        def _f(r, _):
            for q in range(_DH // 16):
                rows0[r, pl.ds(q * 16, 16)] = jnp.zeros((16,), jnp.float32)
            return 0
        lax.fori_loop(0, _K, _f, 0)

    def _zero_acc():
        for q in range(_RPS // _K):
            pltpu.sync_copy(rows0, acc.at[pl.ds(base_r + q * _K, _K)])

    _fill_zero()
    _zero_acc()
    pltpu.sync_copy(src_hbm.at[pl.ds(t * _CPT, _CPT)], sidx)
    pltpu.sync_copy(dst_hbm.at[pl.ds(t * _CPT, _CPT)], didx)
    plsc.subcore_barrier()

    def _phase(hs_hbm, ph):
        def _fire_g(group, s0):
            for i in range(_GRP):
                pltpu.async_copy(hs_hbm.at[sidx.at[group * _GRP + i]],
                                 rows[s0 * _GRP + i], gsem)

        def _drain_g():
            for _i in range(_GRP):
                pltpu.make_async_copy(hs_hbm.at[sidx.at[0]], rows0,
                                      gsem).wait()

        def _fire_s(group, s0):
            for i in range(_GRP):
                pltpu.async_copy(rows[s0 * _GRP + i],
                                 acc.at[didx.at[group * _GRP + i]], ssem,
                                 add=True)

        def _drain_s():
            for _i in range(_GRP):
                pltpu.make_async_copy(rows0, acc.at[didx.at[0]], ssem).wait()

        def _pair(j, last):
            a = 2 * j
            b = a + 1
            _drain_g()
            _fire_s(a, 0)
            _fire_g(b, 1)
            _drain_s()
            _drain_g()
            _fire_s(b, 1)
            if not last:
                _fire_g(a + 2, 0)
            _drain_s()

        _fire_g(0, 0)

        def _body(j, _):
            _pair(j, False)
            return 0
        lax.fori_loop(0, _NGRP // 2 - 1, _body, 0)
        _pair(_NGRP // 2 - 1, True)
        plsc.subcore_barrier()
        pltpu.sync_copy(acc.at[pl.ds(base_r, _RPS)],
                        out_hbm.at[c, ph, pl.ds(base_r, _RPS)])

    _phase(hsa_hbm, 0)
    _fill_zero()
    _zero_acc()
    plsc.subcore_barrier()
    _phase(hsb_hbm, 1)


def _make_scatter_call():
    mesh = plsc.VectorSubcoreMesh(core_axis_name="c", subcore_axis_name="s")
    return pl.kernel(
        _scatter_body,
        out_type=jax.ShapeDtypeStruct((_NC, 2, _NP, _DH), jnp.float32),
        mesh=mesh,
        scratch_types=[
            pltpu.VMEM((_CPT, _K), jnp.int32),
            pltpu.VMEM((_CPT, _K), jnp.int32),
            pltpu.VMEM((_K, _DH), jnp.float32),
            pltpu.VMEM((_K, _DH), jnp.float32),
            pltpu.VMEM((_K, _DH), jnp.float32),
            pltpu.VMEM((_K, _DH), jnp.float32),
            pltpu.VMEM((_K, _DH), jnp.float32),
            pltpu.VMEM((_K, _DH), jnp.float32),
            pltpu.VMEM((_K, _DH), jnp.float32),
            pltpu.VMEM((_K, _DH), jnp.float32),
            pltpu.VMEM_SHARED((_NP, _DH), jnp.float32),
            pltpu.SemaphoreType.DMA,
            pltpu.SemaphoreType.DMA,
        ],
        compiler_params=pltpu.CompilerParams(use_tc_tiling_on_sc=False),
        name="sc_edge_scatter_add",
    )


def _psum_hs(p_ref, hsa_ref, hsb_ref):
    pa = p_ref[0, 0, : _N, :] + p_ref[1, 0, : _N, :] + hsa_ref[...]
    pb = p_ref[0, 1, : _N, :] + p_ref[1, 1, : _N, :] + hsb_ref[...]
    return jnp.concatenate([pa, pb], axis=1)


def _embed_body(x_ref, wx_ref, bx_ref, wg_ref, degs_ref,
                hsa_ref, hsb_ref, dinv_ref):
    deg = degs_ref[0, : _N, 0:1] + degs_ref[1, : _N, 0:1] + 1.0
    dinv = lax.rsqrt(deg)
    h0 = jnp.dot(x_ref[...], wx_ref[...],
                 preferred_element_type=jnp.float32) + bx_ref[...]
    hs = jnp.dot(h0, wg_ref[...], preferred_element_type=jnp.float32) * dinv
    hsa_ref[...] = hs[:, : _DH]
    hsb_ref[...] = hs[:, _DH:]
    dinv_ref[...] = dinv


_embed_call = pl.pallas_call(
    _embed_body,
    out_shape=[
        jax.ShapeDtypeStruct((_N, _DH), jnp.float32),
        jax.ShapeDtypeStruct((_N, _DH), jnp.float32),
        jax.ShapeDtypeStruct((_N, 1), jnp.float32),
    ],
)


def _mid_body(p_ref, hsa_ref, hsb_ref, dinv_ref, bg_ref, g_ref, be_ref,
              wn_ref, outa_ref, outb_ref):
    dinv = dinv_ref[...]
    t = dinv * _psum_hs(p_ref, hsa_ref, hsb_ref) + bg_ref[...]
    m = jnp.mean(t, axis=0, keepdims=True)
    d = t - m
    v = jnp.mean(d * d, axis=0, keepdims=True)
    h = d * lax.rsqrt(v + 1e-5) * g_ref[...] + be_ref[...]
    h = jnp.maximum(h, 0.0)
    hs = jnp.dot(h, wn_ref[...], preferred_element_type=jnp.float32) * dinv
    outa_ref[...] = hs[:, : _DH]
    outb_ref[...] = hs[:, _DH:]


_mid_call = pl.pallas_call(
    _mid_body,
    out_shape=[
        jax.ShapeDtypeStruct((_N, _DH), jnp.float32),
        jax.ShapeDtypeStruct((_N, _DH), jnp.float32),
    ],
)


def _head_body(p_ref, hsa_ref, hsb_ref, dinv_ref, bg_ref, g_ref, be_ref,
               batch_ref, wf0_ref, bf0_ref, wf1_ref, bf1_ref, wf2_ref,
               bf2_ref, out_ref):
    t = dinv_ref[...] * _psum_hs(p_ref, hsa_ref, hsb_ref) + bg_ref[...]
    m = jnp.mean(t, axis=0, keepdims=True)
    d = t - m
    v = jnp.mean(d * d, axis=0, keepdims=True)
    h = d * lax.rsqrt(v + 1e-5) * g_ref[...] + be_ref[...]

    gids = lax.broadcasted_iota(jnp.int32, (_G, 1), 0)
    onehot = (batch_ref[...] == gids).astype(jnp.float32)
    sums = jnp.dot(onehot, h, preferred_element_type=jnp.float32)
    cnt = jnp.sum(onehot, axis=1, keepdims=True)
    o = sums / jnp.maximum(cnt, 1.0)

    o = jnp.maximum(jnp.dot(o, wf0_ref[...],
                            preferred_element_type=jnp.float32) + bf0_ref[...], 0.0)
    o = jnp.maximum(jnp.dot(o, wf1_ref[...],
                            preferred_element_type=jnp.float32) + bf1_ref[...], 0.0)
    out_ref[...] = jnp.dot(o, wf2_ref[...],
                           preferred_element_type=jnp.float32) + bf2_ref[...]


_head_call = pl.pallas_call(
    _head_body,
    out_shape=jax.ShapeDtypeStruct((_G, 1), jnp.float32),
)


def kernel(x, edge_index, edge_attr, batch, Wx, bx, Wg0, bg0, g0, be0,
           Wg1, bg1, g1, be1, Wg2, bg2, g2, be2, Wf0, bf0, Wf1, bf1, Wf2, bf2):
    e = edge_index.shape[1]
    src = edge_index[0].astype(jnp.int32)
    dst = edge_index[1].astype(jnp.int32)
    pad = _EP - e
    pidx = jnp.arange(pad, dtype=jnp.int32)
    src_p = jnp.concatenate([src, pidx % _N])
    dst_p = jnp.concatenate([dst, _N + pidx % (_NP - _N)])
    src_p = src_p.reshape(_NW * _CPT, _K)
    dst_p = dst_p.reshape(_NW * _CPT, _K)

    deg_call = _make_deg_call()
    scatter_call = _make_scatter_call()

    degs = deg_call(dst_p)
    hs0a, hs0b, dinv = _embed_call(x, Wx, bx, Wg0, degs)
    p0 = scatter_call(hs0a, hs0b, src_p, dst_p)
    hs1a, hs1b = _mid_call(p0, hs0a, hs0b, dinv, bg0, g0, be0, Wg1)
    p1 = scatter_call(hs1a, hs1b, src_p, dst_p)
    hs2a, hs2b = _mid_call(p1, hs1a, hs1b, dinv, bg1, g1, be1, Wg2)
    p2 = scatter_call(hs2a, hs2b, src_p, dst_p)
    batch2d = batch.astype(jnp.int32).reshape(1, _N)
    return _head_call(p2, hs2a, hs2b, dinv, bg2, g2, be2, batch2d,
                      Wf0, bf0, Wf1, bf1, Wf2, bf2)
